# baseline (speedup 1.0000x reference)
import numpy as _np

import jax
import jax.numpy as jnp
from jax import lax
from jax.experimental import pallas as pl
from jax.experimental.pallas import tpu as pltpu
from jax.experimental.pallas import tpu_sc as plsc

N_NODES = 10000
N_EDGES = 320000
D_FEAT = 128
D_EDGE = 16
D_OUT = 128

NC = 2
NS = 16
NW = NC * NS
E_PER_W = N_EDGES // NW
CHUNK = 80
GPC = 25
NGRP = E_PER_W // (CHUNK * GPC)
N_PAD = 10240
ROWS_PER_TILE = N_PAD // NS

_SLO = _np.concatenate([_np.arange(32 * g, 32 * g + 16) for g in range(4)])
_SHI = _np.concatenate([_np.arange(32 * g + 16, 32 * g + 32) for g in range(4)])
_SPERM = _np.concatenate([_SLO, _SHI]).astype(_np.int32)
QS = 1024.0


def _xm_body(x_ref, w_ref, b_ref, o_ref):
    o_ref[...] = (
        jnp.dot(x_ref[...], w_ref[...], preferred_element_type=jnp.float32)
        + b_ref[...]
    )


def _em_body(ea_ref, w_ref, o_ref):
    a = jnp.dot(ea_ref[...], w_ref[:, :64], preferred_element_type=jnp.float32)
    b = jnp.dot(ea_ref[...], w_ref[:, 64:], preferred_element_type=jnp.float32)
    ai = (a * QS).astype(jnp.int32)
    bi = (b * QS).astype(jnp.int32)
    o_ref[...] = (ai & jnp.int32(0xFFFF)) | lax.shift_left(bi, 16)


def _upd_body(x_ref, a_ref, w1_ref, w2_ref, b_ref, o_ref):
    agg = a_ref[0] + a_ref[1]
    t = (
        jnp.dot(x_ref[...], w1_ref[...], preferred_element_type=jnp.float32)
        + jnp.dot(agg, w2_ref[...], preferred_element_type=jnp.float32)
        + b_ref[...]
    )
    o_ref[...] = jnp.maximum(t, 0.0)


def _sc_body(xm_hbm, em_hbm, src_hbm, dst_hbm, out_hbm,
             isb, idb, rows, emb, agg_sh, sem_g, sem_e):
    c = lax.axis_index("c")
    s = lax.axis_index("s")
    w = s * NC + c

    zero16 = jnp.zeros((16,), jnp.float32)

    def zrow(i, carry):
        for j in range(8):
            rows[i, pl.ds(j * 16, 16)] = zero16
        return carry

    lax.fori_loop(0, CHUNK, zrow, 0, unroll=False)
    for r in range(ROWS_PER_TILE // CHUNK):
        pltpu.sync_copy(rows, agg_sh.at[pl.ds(s * ROWS_PER_TILE + r * CHUNK, CHUNK)])
    plsc.subcore_barrier()

    sh16 = jnp.full((16,), 16, jnp.int32)
    inv = jnp.full((16,), 1.0 / QS, jnp.float32)
    zf = jnp.zeros((16,), jnp.float32)

    def group(g, carry):
        pltpu.sync_copy(src_hbm.at[w, g], isb)
        pltpu.sync_copy(dst_hbm.at[w, g], idb)

        def chunk(cc, carry2):
            base = w * E_PER_W + (g * GPC + cc) * CHUNK
            pltpu.async_copy(xm_hbm.at[isb.at[2 * cc]],
                             rows.at[pl.ds(0, CHUNK // 2)], sem_g)
            pltpu.async_copy(xm_hbm.at[isb.at[2 * cc + 1]],
                             rows.at[pl.ds(CHUNK // 2, CHUNK // 2)], sem_g)
            e = pltpu.async_copy(em_hbm.at[pl.ds(base, CHUNK)], emb, sem_e)
            pltpu.make_async_copy(xm_hbm.at[isb.at[2 * cc]],
                                  rows.at[pl.ds(0, CHUNK // 2)], sem_g).wait()
            pltpu.make_async_copy(xm_hbm.at[isb.at[2 * cc + 1]],
                                  rows.at[pl.ds(CHUNK // 2, CHUNK // 2)],
                                  sem_g).wait()
            e.wait()

            def erow(r, carry3):
                for gg in range(D_OUT // 32):
                    we = emb[r, pl.ds(gg * 16, 16)]
                    elo = lax.shift_right_arithmetic(
                        lax.shift_left(we, sh16), sh16).astype(jnp.float32) * inv
                    ehi = lax.shift_right_arithmetic(
                        we, sh16).astype(jnp.float32) * inv
                    lo_sl = pl.ds(gg * 32, 16)
                    hi_sl = pl.ds(gg * 32 + 16, 16)
                    rows[r, lo_sl] = jnp.maximum(rows[r, lo_sl] + elo, zf)
                    rows[r, hi_sl] = jnp.maximum(rows[r, hi_sl] + ehi, zf)
                return carry3

            lax.fori_loop(0, CHUNK, erow, 0, unroll=False)
            pltpu.sync_copy(rows, agg_sh.at[idb.at[cc]], add=True)
            return carry2

        lax.fori_loop(0, GPC, chunk, 0, unroll=False)
        return carry

    lax.fori_loop(0, NGRP, group, 0, unroll=False)

    plsc.subcore_barrier()
    off = s * ROWS_PER_TILE
    pltpu.sync_copy(agg_sh.at[pl.ds(off, ROWS_PER_TILE)],
                    out_hbm.at[c, pl.ds(off, ROWS_PER_TILE)])


@jax.jit
def _run(x, src, dst, edge_attr, W_msg, b_msg, W_upd, b_upd):
    perm = jnp.asarray(_SPERM)
    W1 = W_msg[:D_FEAT]
    W2 = W_msg[D_FEAT:][:, perm]
    Wu1 = W_upd[:D_FEAT]
    Wu2 = W_upd[D_FEAT:]
    b_msg2 = b_msg.reshape(1, D_OUT)
    b_upd2 = b_upd.reshape(1, D_OUT)

    xm = pl.pallas_call(
        _xm_body,
        out_shape=jax.ShapeDtypeStruct((N_NODES, D_OUT), jnp.float32),
    )(x, W1, b_msg2)

    EB = 8000
    em = pl.pallas_call(
        _em_body,
        grid=(N_EDGES // EB,),
        in_specs=[
            pl.BlockSpec((EB, D_EDGE), lambda i: (i, 0)),
            pl.BlockSpec((D_EDGE, D_OUT), lambda i: (0, 0)),
        ],
        out_specs=pl.BlockSpec((EB, D_OUT // 2), lambda i: (i, 0)),
        out_shape=jax.ShapeDtypeStruct((N_EDGES, D_OUT // 2), jnp.int32),
    )(edge_attr, W2)

    mesh = plsc.VectorSubcoreMesh(
        core_axis_name="c", subcore_axis_name="s", num_cores=NC, num_subcores=NS
    )
    agg2 = pl.kernel(
        _sc_body,
        out_type=jax.ShapeDtypeStruct((NC, N_PAD, D_OUT), jnp.float32),
        mesh=mesh,
        scratch_types=[
            pltpu.VMEM((2 * GPC, CHUNK // 2), jnp.int32),
            pltpu.VMEM((GPC, CHUNK), jnp.int32),
            pltpu.VMEM((CHUNK, D_OUT), jnp.float32),
            pltpu.VMEM((CHUNK, D_OUT // 2), jnp.int32),
            pltpu.VMEM_SHARED((N_PAD, D_OUT), jnp.float32),
            pltpu.SemaphoreType.DMA,
            pltpu.SemaphoreType.DMA,
        ],
    )(xm, em, src.reshape(NW, NGRP, 2 * GPC, CHUNK // 2),
      dst.reshape(NW, NGRP, GPC, CHUNK))
    agg2 = agg2[:, :N_NODES]

    NB = 2000
    out = pl.pallas_call(
        _upd_body,
        grid=(N_NODES // NB,),
        in_specs=[
            pl.BlockSpec((NB, D_FEAT), lambda i: (i, 0)),
            pl.BlockSpec((NC, NB, D_OUT), lambda i: (0, i, 0)),
            pl.BlockSpec((D_FEAT, D_OUT), lambda i: (0, 0)),
            pl.BlockSpec((D_OUT, D_OUT), lambda i: (0, 0)),
            pl.BlockSpec((1, D_OUT), lambda i: (0, 0)),
        ],
        out_specs=pl.BlockSpec((NB, D_OUT), lambda i: (i, 0)),
        out_shape=jax.ShapeDtypeStruct((N_NODES, D_OUT), jnp.float32),
    )(x, agg2, Wu1, Wu2, b_upd2)
    return out


def kernel(x, edge_index, edge_attr, W_msg, b_msg, W_upd, b_upd):
    src = edge_index[0].astype(jnp.int32)
    dst = edge_index[1].astype(jnp.int32)
    return _run(x, src, dst, edge_attr, W_msg, b_msg, W_upd, b_upd)


# speedup vs baseline: 1.0078x; 1.0078x over previous
import numpy as _np

import jax
import jax.numpy as jnp
from jax import lax
from jax.experimental import pallas as pl
from jax.experimental.pallas import tpu as pltpu
from jax.experimental.pallas import tpu_sc as plsc

N_NODES = 10000
N_EDGES = 320000
D_FEAT = 128
D_EDGE = 16
D_OUT = 128

NC = 2
NS = 16
NW = NC * NS
E_PER_W = N_EDGES // NW
CHUNK = 80
GPC = 25
NGRP = E_PER_W // (CHUNK * GPC)
N_PAD = 10240
ROWS_PER_TILE = N_PAD // NS
NSTR = 4

_SLO = _np.concatenate([_np.arange(32 * g, 32 * g + 16) for g in range(4)])
_SHI = _np.concatenate([_np.arange(32 * g + 16, 32 * g + 32) for g in range(4)])
_SPERM = _np.concatenate([_SLO, _SHI]).astype(_np.int32)
QS = 1024.0


def _xm_body(x_ref, w_ref, b_ref, o_ref):
    o_ref[...] = (
        jnp.dot(x_ref[...], w_ref[...], preferred_element_type=jnp.float32)
        + b_ref[...]
    )


def _em_body(ea_ref, w_ref, o_ref):
    a = jnp.dot(ea_ref[...], w_ref[:, :64], preferred_element_type=jnp.float32)
    b = jnp.dot(ea_ref[...], w_ref[:, 64:], preferred_element_type=jnp.float32)
    ai = (a * QS).astype(jnp.int32)
    bi = (b * QS).astype(jnp.int32)
    o_ref[...] = (ai & jnp.int32(0xFFFF)) | lax.shift_left(bi, 16)


def _upd_body(x_ref, a_ref, w1_ref, w2_ref, b_ref, o_ref):
    agg = a_ref[0] + a_ref[1]
    t = (
        jnp.dot(x_ref[...], w1_ref[...], preferred_element_type=jnp.float32)
        + jnp.dot(agg, w2_ref[...], preferred_element_type=jnp.float32)
        + b_ref[...]
    )
    o_ref[...] = jnp.maximum(t, 0.0)


def _sc_body(xm_hbm, em_hbm, src_hbm, dst_hbm, out_hbm,
             isb, idb, rows, emb, agg_sh, sem_g, sem_e):
    c = lax.axis_index("c")
    s = lax.axis_index("s")
    w = s * NC + c

    zero16 = jnp.zeros((16,), jnp.float32)

    def zrow(i, carry):
        for j in range(8):
            rows[i, pl.ds(j * 16, 16)] = zero16
        return carry

    lax.fori_loop(0, CHUNK, zrow, 0, unroll=False)
    for r in range(ROWS_PER_TILE // CHUNK):
        pltpu.sync_copy(rows, agg_sh.at[pl.ds(s * ROWS_PER_TILE + r * CHUNK, CHUNK)])
    plsc.subcore_barrier()

    sh16 = jnp.full((16,), 16, jnp.int32)
    inv = jnp.full((16,), 1.0 / QS, jnp.float32)
    zf = jnp.zeros((16,), jnp.float32)

    def group(g, carry):
        pltpu.sync_copy(src_hbm.at[w, g], isb)
        pltpu.sync_copy(dst_hbm.at[w, g], idb)

        def chunk(cc, carry2):
            base = w * E_PER_W + (g * GPC + cc) * CHUNK
            for t in range(NSTR):
                pltpu.async_copy(
                    xm_hbm.at[isb.at[NSTR * cc + t]],
                    rows.at[pl.ds(t * (CHUNK // NSTR), CHUNK // NSTR)], sem_g)
            e = pltpu.async_copy(em_hbm.at[pl.ds(base, CHUNK)], emb, sem_e)
            for t in range(NSTR):
                pltpu.make_async_copy(
                    xm_hbm.at[isb.at[NSTR * cc + t]],
                    rows.at[pl.ds(t * (CHUNK // NSTR), CHUNK // NSTR)],
                    sem_g).wait()
            e.wait()

            def erow(r, carry3):
                for gg in range(D_OUT // 32):
                    we = emb[r, pl.ds(gg * 16, 16)]
                    elo = lax.shift_right_arithmetic(
                        lax.shift_left(we, sh16), sh16).astype(jnp.float32) * inv
                    ehi = lax.shift_right_arithmetic(
                        we, sh16).astype(jnp.float32) * inv
                    lo_sl = pl.ds(gg * 32, 16)
                    hi_sl = pl.ds(gg * 32 + 16, 16)
                    rows[r, lo_sl] = jnp.maximum(rows[r, lo_sl] + elo, zf)
                    rows[r, hi_sl] = jnp.maximum(rows[r, hi_sl] + ehi, zf)
                return carry3

            lax.fori_loop(0, CHUNK, erow, 0, unroll=False)
            pltpu.sync_copy(rows, agg_sh.at[idb.at[cc]], add=True)
            return carry2

        lax.fori_loop(0, GPC, chunk, 0, unroll=False)
        return carry

    lax.fori_loop(0, NGRP, group, 0, unroll=False)

    plsc.subcore_barrier()
    off = s * ROWS_PER_TILE
    pltpu.sync_copy(agg_sh.at[pl.ds(off, ROWS_PER_TILE)],
                    out_hbm.at[c, pl.ds(off, ROWS_PER_TILE)])


@jax.jit
def _run(x, src, dst, edge_attr, W_msg, b_msg, W_upd, b_upd):
    perm = jnp.asarray(_SPERM)
    W1 = W_msg[:D_FEAT]
    W2 = W_msg[D_FEAT:][:, perm]
    Wu1 = W_upd[:D_FEAT]
    Wu2 = W_upd[D_FEAT:]
    b_msg2 = b_msg.reshape(1, D_OUT)
    b_upd2 = b_upd.reshape(1, D_OUT)

    xm = pl.pallas_call(
        _xm_body,
        out_shape=jax.ShapeDtypeStruct((N_NODES, D_OUT), jnp.float32),
    )(x, W1, b_msg2)

    EB = 8000
    em = pl.pallas_call(
        _em_body,
        grid=(N_EDGES // EB,),
        in_specs=[
            pl.BlockSpec((EB, D_EDGE), lambda i: (i, 0)),
            pl.BlockSpec((D_EDGE, D_OUT), lambda i: (0, 0)),
        ],
        out_specs=pl.BlockSpec((EB, D_OUT // 2), lambda i: (i, 0)),
        out_shape=jax.ShapeDtypeStruct((N_EDGES, D_OUT // 2), jnp.int32),
    )(edge_attr, W2)

    mesh = plsc.VectorSubcoreMesh(
        core_axis_name="c", subcore_axis_name="s", num_cores=NC, num_subcores=NS
    )
    agg2 = pl.kernel(
        _sc_body,
        out_type=jax.ShapeDtypeStruct((NC, N_PAD, D_OUT), jnp.float32),
        mesh=mesh,
        scratch_types=[
            pltpu.VMEM((NSTR * GPC, CHUNK // NSTR), jnp.int32),
            pltpu.VMEM((GPC, CHUNK), jnp.int32),
            pltpu.VMEM((CHUNK, D_OUT), jnp.float32),
            pltpu.VMEM((CHUNK, D_OUT // 2), jnp.int32),
            pltpu.VMEM_SHARED((N_PAD, D_OUT), jnp.float32),
            pltpu.SemaphoreType.DMA,
            pltpu.SemaphoreType.DMA,
        ],
    )(xm, em, src.reshape(NW, NGRP, NSTR * GPC, CHUNK // NSTR),
      dst.reshape(NW, NGRP, GPC, CHUNK))

    NB = 2000
    out = pl.pallas_call(
        _upd_body,
        grid=(N_NODES // NB,),
        in_specs=[
            pl.BlockSpec((NB, D_FEAT), lambda i: (i, 0)),
            pl.BlockSpec((NC, NB, D_OUT), lambda i: (0, i, 0)),
            pl.BlockSpec((D_FEAT, D_OUT), lambda i: (0, 0)),
            pl.BlockSpec((D_OUT, D_OUT), lambda i: (0, 0)),
            pl.BlockSpec((1, D_OUT), lambda i: (0, 0)),
        ],
        out_specs=pl.BlockSpec((NB, D_OUT), lambda i: (i, 0)),
        out_shape=jax.ShapeDtypeStruct((N_NODES, D_OUT), jnp.float32),
    )(x, agg2, Wu1, Wu2, b_upd2)
    return out


def kernel(x, edge_index, edge_attr, W_msg, b_msg, W_upd, b_upd):
    src = edge_index[0].astype(jnp.int32)
    dst = edge_index[1].astype(jnp.int32)
    return _run(x, src, dst, edge_attr, W_msg, b_msg, W_upd, b_upd)


# speedup vs baseline: 1.0329x; 1.0249x over previous
import numpy as _np

import jax
import jax.numpy as jnp
from jax import lax
from jax.experimental import pallas as pl
from jax.experimental.pallas import tpu as pltpu
from jax.experimental.pallas import tpu_sc as plsc

N_NODES = 10000
N_EDGES = 320000
D_FEAT = 128
D_EDGE = 16
D_OUT = 128

NC = 2
NS = 16
NW = NC * NS
E_PER_W = N_EDGES // NW
CHUNK = 80
GPC = 25
NGRP = E_PER_W // (CHUNK * GPC)
N_PAD = 10240
ROWS_PER_TILE = N_PAD // NS
NSTR = 1

_SLO = _np.concatenate([_np.arange(32 * g, 32 * g + 16) for g in range(4)])
_SHI = _np.concatenate([_np.arange(32 * g + 16, 32 * g + 32) for g in range(4)])
_SPERM = _np.concatenate([_SLO, _SHI]).astype(_np.int32)
QS = 1024.0


def _xm_body(x_ref, w_ref, b_ref, o_ref):
    o_ref[...] = (
        jnp.dot(x_ref[...], w_ref[...], preferred_element_type=jnp.float32)
        + b_ref[...]
    )


def _em_body(ea_ref, w_ref, o_ref):
    a = jnp.dot(ea_ref[...], w_ref[:, :64], preferred_element_type=jnp.float32)
    b = jnp.dot(ea_ref[...], w_ref[:, 64:], preferred_element_type=jnp.float32)
    ai = (a * QS).astype(jnp.int32)
    bi = (b * QS).astype(jnp.int32)
    o_ref[...] = (ai & jnp.int32(0xFFFF)) | lax.shift_left(bi, 16)


def _upd_body(x_ref, a_ref, w1_ref, w2_ref, b_ref, o_ref):
    agg = a_ref[0] + a_ref[1]
    t = (
        jnp.dot(x_ref[...], w1_ref[...], preferred_element_type=jnp.float32)
        + jnp.dot(agg, w2_ref[...], preferred_element_type=jnp.float32)
        + b_ref[...]
    )
    o_ref[...] = jnp.maximum(t, 0.0)


def _sc_body(xm_hbm, em_hbm, src_hbm, dst_hbm, out_hbm,
             isb, idb, rows, emb, agg_sh, sem_d, sem_g, sem_e):
    c = lax.axis_index("c")
    s = lax.axis_index("s")
    w = s * NC + c

    zero16 = jnp.zeros((16,), jnp.float32)

    def zrow(i, carry):
        for j in range(8):
            rows[i, pl.ds(j * 16, 16)] = zero16
        return carry

    lax.fori_loop(0, CHUNK, zrow, 0, unroll=False)
    for r in range(ROWS_PER_TILE // CHUNK):
        pltpu.sync_copy(rows, agg_sh.at[pl.ds(s * ROWS_PER_TILE + r * CHUNK, CHUNK)])
    plsc.subcore_barrier()

    sh16 = jnp.full((16,), 16, jnp.int32)
    inv = jnp.full((16,), 1.0 / QS, jnp.float32)
    zf = jnp.zeros((16,), jnp.float32)

    def group(g, carry):
        gbase = w * E_PER_W + g * GPC * CHUNK
        pltpu.sync_copy(src_hbm.at[pl.ds(gbase, GPC * CHUNK)], isb)

        def chunk(cc, carry2):
            base = w * E_PER_W + (g * GPC + cc) * CHUNK
            d = pltpu.async_copy(dst_hbm.at[pl.ds(base, CHUNK)], idb, sem_d)
            pltpu.async_copy(xm_hbm.at[isb.at[pl.ds(cc * CHUNK, CHUNK)]],
                             rows, sem_g)
            e = pltpu.async_copy(em_hbm.at[pl.ds(base, CHUNK)], emb, sem_e)
            pltpu.make_async_copy(xm_hbm.at[isb.at[pl.ds(cc * CHUNK, CHUNK)]],
                                  rows, sem_g).wait()
            e.wait()

            def erow(r, carry3):
                for gg in range(D_OUT // 32):
                    we = emb[r, pl.ds(gg * 16, 16)]
                    elo = lax.shift_right_arithmetic(
                        lax.shift_left(we, sh16), sh16).astype(jnp.float32) * inv
                    ehi = lax.shift_right_arithmetic(
                        we, sh16).astype(jnp.float32) * inv
                    lo_sl = pl.ds(gg * 32, 16)
                    hi_sl = pl.ds(gg * 32 + 16, 16)
                    rows[r, lo_sl] = jnp.maximum(rows[r, lo_sl] + elo, zf)
                    rows[r, hi_sl] = jnp.maximum(rows[r, hi_sl] + ehi, zf)
                return carry3

            lax.fori_loop(0, CHUNK, erow, 0, unroll=False)
            d.wait()
            pltpu.sync_copy(rows, agg_sh.at[idb], add=True)
            return carry2

        lax.fori_loop(0, GPC, chunk, 0, unroll=False)
        return carry

    lax.fori_loop(0, NGRP, group, 0, unroll=False)

    plsc.subcore_barrier()
    off = s * ROWS_PER_TILE
    pltpu.sync_copy(agg_sh.at[pl.ds(off, ROWS_PER_TILE)],
                    out_hbm.at[c, pl.ds(off, ROWS_PER_TILE)])


@jax.jit
def _run(x, src, dst, edge_attr, W_msg, b_msg, W_upd, b_upd):
    perm = jnp.asarray(_SPERM)
    W1 = W_msg[:D_FEAT]
    W2 = W_msg[D_FEAT:][:, perm]
    Wu1 = W_upd[:D_FEAT]
    Wu2 = W_upd[D_FEAT:]
    b_msg2 = b_msg.reshape(1, D_OUT)
    b_upd2 = b_upd.reshape(1, D_OUT)

    xm = pl.pallas_call(
        _xm_body,
        out_shape=jax.ShapeDtypeStruct((N_NODES, D_OUT), jnp.float32),
    )(x, W1, b_msg2)

    EB = 16000
    em = pl.pallas_call(
        _em_body,
        grid=(N_EDGES // EB,),
        in_specs=[
            pl.BlockSpec((EB, D_EDGE), lambda i: (i, 0)),
            pl.BlockSpec((D_EDGE, D_OUT), lambda i: (0, 0)),
        ],
        out_specs=pl.BlockSpec((EB, D_OUT // 2), lambda i: (i, 0)),
        out_shape=jax.ShapeDtypeStruct((N_EDGES, D_OUT // 2), jnp.int32),
    )(edge_attr, W2)

    mesh = plsc.VectorSubcoreMesh(
        core_axis_name="c", subcore_axis_name="s", num_cores=NC, num_subcores=NS
    )
    agg2 = pl.kernel(
        _sc_body,
        out_type=jax.ShapeDtypeStruct((NC, N_PAD, D_OUT), jnp.float32),
        mesh=mesh,
        scratch_types=[
            pltpu.VMEM((GPC * CHUNK,), jnp.int32),
            pltpu.VMEM((CHUNK,), jnp.int32),
            pltpu.VMEM((CHUNK, D_OUT), jnp.float32),
            pltpu.VMEM((CHUNK, D_OUT // 2), jnp.int32),
            pltpu.VMEM_SHARED((N_PAD, D_OUT), jnp.float32),
            pltpu.SemaphoreType.DMA,
            pltpu.SemaphoreType.DMA,
            pltpu.SemaphoreType.DMA,
        ],
    )(xm, em, src, dst)

    NB = 2000
    out = pl.pallas_call(
        _upd_body,
        grid=(N_NODES // NB,),
        in_specs=[
            pl.BlockSpec((NB, D_FEAT), lambda i: (i, 0)),
            pl.BlockSpec((NC, NB, D_OUT), lambda i: (0, i, 0)),
            pl.BlockSpec((D_FEAT, D_OUT), lambda i: (0, 0)),
            pl.BlockSpec((D_OUT, D_OUT), lambda i: (0, 0)),
            pl.BlockSpec((1, D_OUT), lambda i: (0, 0)),
        ],
        out_specs=pl.BlockSpec((NB, D_OUT), lambda i: (i, 0)),
        out_shape=jax.ShapeDtypeStruct((N_NODES, D_OUT), jnp.float32),
    )(x, agg2, Wu1, Wu2, b_upd2)
    return out


def kernel(x, edge_index, edge_attr, W_msg, b_msg, W_upd, b_upd):
    src = edge_index[0].astype(jnp.int32)
    dst = edge_index[1].astype(jnp.int32)
    return _run(x, src, dst, edge_attr, W_msg, b_msg, W_upd, b_upd)


# speedup vs baseline: 1.3191x; 1.2771x over previous
import numpy as _np

import jax
import jax.numpy as jnp
from jax import lax
from jax.experimental import pallas as pl
from jax.experimental.pallas import tpu as pltpu
from jax.experimental.pallas import tpu_sc as plsc

N_NODES = 10000
N_EDGES = 320000
D_FEAT = 128
D_EDGE = 16
D_OUT = 128

NC = 2
NS = 16
NW = NC * NS
E_PER_W = N_EDGES // NW
CHUNK = 80
GPC = 25
NGRP = E_PER_W // (CHUNK * GPC)
N_PAD = 10240
ROWS_PER_TILE = N_PAD // NS
NSTR = 1

_SLO = _np.concatenate([_np.arange(32 * g, 32 * g + 16) for g in range(4)])
_SHI = _np.concatenate([_np.arange(32 * g + 16, 32 * g + 32) for g in range(4)])
_SPERM = _np.concatenate([_SLO, _SHI]).astype(_np.int32)
QS = 1024.0


def _xm_body(x_ref, w_ref, b_ref, o_ref):
    o_ref[...] = (
        jnp.dot(x_ref[...], w_ref[...], preferred_element_type=jnp.float32)
        + b_ref[...]
    )


def _em_body(ea_ref, w_ref, o_ref):
    dn = (((0,), (0,)), ((), ()))
    a = lax.dot_general(ea_ref[...], w_ref[:, :64], dn,
                        preferred_element_type=jnp.float32)
    b = lax.dot_general(ea_ref[...], w_ref[:, 64:], dn,
                        preferred_element_type=jnp.float32)
    ai = (a * QS).astype(jnp.int32)
    bi = (b * QS).astype(jnp.int32)
    o_ref[...] = (ai & jnp.int32(0xFFFF)) | lax.shift_left(bi, 16)


def _upd_body(x_ref, a_ref, w1_ref, w2_ref, b_ref, o_ref):
    agg = a_ref[0] + a_ref[1]
    t = (
        jnp.dot(x_ref[...], w1_ref[...], preferred_element_type=jnp.float32)
        + jnp.dot(agg, w2_ref[...], preferred_element_type=jnp.float32)
        + b_ref[...]
    )
    o_ref[...] = jnp.maximum(t, 0.0)


def _sc_body(xm_hbm, em_hbm, src_hbm, dst_hbm, out_hbm,
             isb, idb, rows, emb, agg_sh, sem_d, sem_g, sem_e):
    c = lax.axis_index("c")
    s = lax.axis_index("s")
    w = s * NC + c

    zero16 = jnp.zeros((16,), jnp.float32)

    def zrow(i, carry):
        for j in range(8):
            rows[i, pl.ds(j * 16, 16)] = zero16
        return carry

    lax.fori_loop(0, CHUNK, zrow, 0, unroll=False)
    for r in range(ROWS_PER_TILE // CHUNK):
        pltpu.sync_copy(rows, agg_sh.at[pl.ds(s * ROWS_PER_TILE + r * CHUNK, CHUNK)])
    plsc.subcore_barrier()

    sh16 = jnp.full((16,), 16, jnp.int32)
    inv = jnp.full((16,), 1.0 / QS, jnp.float32)
    zf = jnp.zeros((16,), jnp.float32)

    def group(g, carry):
        gbase = w * E_PER_W + g * GPC * CHUNK
        pltpu.sync_copy(src_hbm.at[pl.ds(gbase, GPC * CHUNK)], isb)

        def chunk(cc, carry2):
            base = w * E_PER_W + (g * GPC + cc) * CHUNK
            d = pltpu.async_copy(dst_hbm.at[pl.ds(base, CHUNK)], idb, sem_d)
            pltpu.async_copy(xm_hbm.at[isb.at[pl.ds(cc * CHUNK, CHUNK)]],
                             rows, sem_g)
            e = pltpu.async_copy(em_hbm.at[pl.ds(base, CHUNK)], emb, sem_e)
            pltpu.make_async_copy(xm_hbm.at[isb.at[pl.ds(cc * CHUNK, CHUNK)]],
                                  rows, sem_g).wait()
            e.wait()

            def erow(r, carry3):
                for gg in range(D_OUT // 32):
                    we = emb[r, pl.ds(gg * 16, 16)]
                    elo = lax.shift_right_arithmetic(
                        lax.shift_left(we, sh16), sh16).astype(jnp.float32) * inv
                    ehi = lax.shift_right_arithmetic(
                        we, sh16).astype(jnp.float32) * inv
                    lo_sl = pl.ds(gg * 32, 16)
                    hi_sl = pl.ds(gg * 32 + 16, 16)
                    rows[r, lo_sl] = jnp.maximum(rows[r, lo_sl] + elo, zf)
                    rows[r, hi_sl] = jnp.maximum(rows[r, hi_sl] + ehi, zf)
                return carry3

            lax.fori_loop(0, CHUNK, erow, 0, unroll=False)
            d.wait()
            pltpu.sync_copy(rows, agg_sh.at[idb], add=True)
            return carry2

        lax.fori_loop(0, GPC, chunk, 0, unroll=False)
        return carry

    lax.fori_loop(0, NGRP, group, 0, unroll=False)

    plsc.subcore_barrier()
    off = s * ROWS_PER_TILE
    pltpu.sync_copy(agg_sh.at[pl.ds(off, ROWS_PER_TILE)],
                    out_hbm.at[c, pl.ds(off, ROWS_PER_TILE)])


@jax.jit
def _run(x, src, dst, edge_attr, W_msg, b_msg, W_upd, b_upd):
    perm = jnp.asarray(_SPERM)
    W1 = W_msg[:D_FEAT]
    W2 = W_msg[D_FEAT:][:, perm]
    Wu1 = W_upd[:D_FEAT]
    Wu2 = W_upd[D_FEAT:]
    b_msg2 = b_msg.reshape(1, D_OUT)
    b_upd2 = b_upd.reshape(1, D_OUT)

    xm = pl.pallas_call(
        _xm_body,
        out_shape=jax.ShapeDtypeStruct((N_NODES, D_OUT), jnp.float32),
    )(x, W1, b_msg2)

    EB = 16000
    em = pl.pallas_call(
        _em_body,
        grid=(N_EDGES // EB,),
        in_specs=[
            pl.BlockSpec((D_EDGE, EB), lambda i: (0, i)),
            pl.BlockSpec((D_EDGE, D_OUT), lambda i: (0, 0)),
        ],
        out_specs=pl.BlockSpec((EB, D_OUT // 2), lambda i: (i, 0)),
        out_shape=jax.ShapeDtypeStruct((N_EDGES, D_OUT // 2), jnp.int32),
    )(edge_attr.T, W2)

    mesh = plsc.VectorSubcoreMesh(
        core_axis_name="c", subcore_axis_name="s", num_cores=NC, num_subcores=NS
    )
    agg2 = pl.kernel(
        _sc_body,
        out_type=jax.ShapeDtypeStruct((NC, N_PAD, D_OUT), jnp.float32),
        mesh=mesh,
        scratch_types=[
            pltpu.VMEM((GPC * CHUNK,), jnp.int32),
            pltpu.VMEM((CHUNK,), jnp.int32),
            pltpu.VMEM((CHUNK, D_OUT), jnp.float32),
            pltpu.VMEM((CHUNK, D_OUT // 2), jnp.int32),
            pltpu.VMEM_SHARED((N_PAD, D_OUT), jnp.float32),
            pltpu.SemaphoreType.DMA,
            pltpu.SemaphoreType.DMA,
            pltpu.SemaphoreType.DMA,
        ],
    )(xm, em, src, dst)

    NB = 2000
    out = pl.pallas_call(
        _upd_body,
        grid=(N_NODES // NB,),
        in_specs=[
            pl.BlockSpec((NB, D_FEAT), lambda i: (i, 0)),
            pl.BlockSpec((NC, NB, D_OUT), lambda i: (0, i, 0)),
            pl.BlockSpec((D_FEAT, D_OUT), lambda i: (0, 0)),
            pl.BlockSpec((D_OUT, D_OUT), lambda i: (0, 0)),
            pl.BlockSpec((1, D_OUT), lambda i: (0, 0)),
        ],
        out_specs=pl.BlockSpec((NB, D_OUT), lambda i: (i, 0)),
        out_shape=jax.ShapeDtypeStruct((N_NODES, D_OUT), jnp.float32),
    )(x, agg2, Wu1, Wu2, b_upd2)
    return out


def kernel(x, edge_index, edge_attr, W_msg, b_msg, W_upd, b_upd):
    src = edge_index[0].astype(jnp.int32)
    dst = edge_index[1].astype(jnp.int32)
    return _run(x, src, dst, edge_attr, W_msg, b_msg, W_upd, b_upd)


# speedup vs baseline: 1.8844x; 1.4285x over previous
import numpy as _np

import jax
import jax.numpy as jnp
from jax import lax
from jax.experimental import pallas as pl
from jax.experimental.pallas import tpu as pltpu
from jax.experimental.pallas import tpu_sc as plsc

N_NODES = 10000
N_EDGES = 320000
D_FEAT = 128
D_EDGE = 16
D_OUT = 128

NC = 2
NS = 16
NW = NC * NS
E_PER_W = N_EDGES // NW
CHUNK = 80
GPC = 25
NGRP = E_PER_W // (CHUNK * GPC)
N_PAD = 10240
ROWS_PER_TILE = N_PAD // NS
NSTR = 1

_SLO = _np.concatenate([_np.arange(32 * g, 32 * g + 16) for g in range(4)])
_SHI = _np.concatenate([_np.arange(32 * g + 16, 32 * g + 32) for g in range(4)])
_SPERM = _np.concatenate([_SLO, _SHI]).astype(_np.int32)
QS = 1024.0


def _xm_body(x_ref, w_ref, b_ref, o_ref):
    o_ref[...] = (
        jnp.dot(x_ref[...], w_ref[...], preferred_element_type=jnp.float32)
        + b_ref[...]
    )


def _em_body(ea_ref, w_ref, o_ref):
    dn = (((0,), (0,)), ((), ()))
    a = lax.dot_general(ea_ref[...], w_ref[:, :64], dn,
                        preferred_element_type=jnp.float32)
    b = lax.dot_general(ea_ref[...], w_ref[:, 64:], dn,
                        preferred_element_type=jnp.float32)
    ai = (a * QS).astype(jnp.int32)
    bi = (b * QS).astype(jnp.int32)
    o_ref[...] = (ai & jnp.int32(0xFFFF)) | lax.shift_left(bi, 16)


def _upd_body(x_ref, a_ref, w1_ref, w2_ref, b_ref, o_ref):
    agg = a_ref[0] + a_ref[1]
    t = (
        jnp.dot(x_ref[...], w1_ref[...], preferred_element_type=jnp.float32)
        + jnp.dot(agg, w2_ref[...], preferred_element_type=jnp.float32)
        + b_ref[...]
    )
    o_ref[...] = jnp.maximum(t, 0.0)


def _sc_body(xm_hbm, em_hbm, src_hbm, dst_hbm, out_hbm,
             isb, idb0, idb1, rows0, rows1, emb0, emb1, agg_sh,
             sem_d0, sem_d1, sem_g0, sem_g1, sem_e0, sem_e1):
    c = lax.axis_index("c")
    s = lax.axis_index("s")
    w = s * NC + c

    zero16 = jnp.zeros((16,), jnp.float32)

    def zrow(i, carry):
        for j in range(8):
            rows0[i, pl.ds(j * 16, 16)] = zero16
        return carry

    lax.fori_loop(0, CHUNK, zrow, 0, unroll=False)
    for r in range(ROWS_PER_TILE // CHUNK):
        pltpu.sync_copy(rows0, agg_sh.at[pl.ds(s * ROWS_PER_TILE + r * CHUNK, CHUNK)])
    plsc.subcore_barrier()

    sh16 = jnp.full((16,), 16, jnp.int32)
    inv = jnp.full((16,), 1.0 / QS, jnp.float32)
    zf = jnp.zeros((16,), jnp.float32)

    rbuf = (rows0, rows1)
    ebuf = (emb0, emb1)
    dbuf = (idb0, idb1)
    dsem = (sem_d0, sem_d1)
    gsem = (sem_g0, sem_g1)
    esem = (sem_e0, sem_e1)

    def group(g, carry):
        gbase = w * E_PER_W + g * GPC * CHUNK
        pltpu.sync_copy(src_hbm.at[pl.ds(gbase, GPC * CHUNK)], isb)

        def start(cc, p):
            base = w * E_PER_W + (g * GPC + cc) * CHUNK
            pltpu.async_copy(dst_hbm.at[pl.ds(base, CHUNK)], dbuf[p], dsem[p])
            pltpu.async_copy(xm_hbm.at[isb.at[pl.ds(cc * CHUNK, CHUNK)]],
                             rbuf[p], gsem[p])
            pltpu.async_copy(em_hbm.at[pl.ds(base, CHUNK)], ebuf[p], esem[p])

        def work(cc, p):
            base = w * E_PER_W + (g * GPC + cc) * CHUNK
            rows = rbuf[p]
            emb = ebuf[p]
            pltpu.make_async_copy(xm_hbm.at[isb.at[pl.ds(cc * CHUNK, CHUNK)]],
                                  rows, gsem[p]).wait()
            pltpu.make_async_copy(em_hbm.at[pl.ds(base, CHUNK)], emb,
                                  esem[p]).wait()

            def erow(r, carry3):
                for gg in range(D_OUT // 32):
                    we = emb[r, pl.ds(gg * 16, 16)]
                    elo = lax.shift_right_arithmetic(
                        lax.shift_left(we, sh16), sh16).astype(jnp.float32) * inv
                    ehi = lax.shift_right_arithmetic(
                        we, sh16).astype(jnp.float32) * inv
                    lo_sl = pl.ds(gg * 32, 16)
                    hi_sl = pl.ds(gg * 32 + 16, 16)
                    rows[r, lo_sl] = jnp.maximum(rows[r, lo_sl] + elo, zf)
                    rows[r, hi_sl] = jnp.maximum(rows[r, hi_sl] + ehi, zf)
                return carry3

            lax.fori_loop(0, CHUNK, erow, 0, unroll=False)
            pltpu.make_async_copy(dst_hbm.at[pl.ds(base, CHUNK)], dbuf[p],
                                  dsem[p]).wait()
            pltpu.sync_copy(rows, agg_sh.at[dbuf[p]], add=True)

        start(0, 0)

        def pair(k, carry2):
            a = 2 * k
            start(a + 1, 1)
            work(a, 0)
            start(a + 2, 0)
            work(a + 1, 1)
            return carry2

        lax.fori_loop(0, (GPC - 1) // 2, pair, 0, unroll=False)
        work(GPC - 1, 0)
        return carry

    lax.fori_loop(0, NGRP, group, 0, unroll=False)

    plsc.subcore_barrier()
    off = s * ROWS_PER_TILE
    pltpu.sync_copy(agg_sh.at[pl.ds(off, ROWS_PER_TILE)],
                    out_hbm.at[c, pl.ds(off, ROWS_PER_TILE)])


@jax.jit
def _run(x, src, dst, edge_attr, W_msg, b_msg, W_upd, b_upd):
    perm = jnp.asarray(_SPERM)
    W1 = W_msg[:D_FEAT]
    W2 = W_msg[D_FEAT:][:, perm]
    Wu1 = W_upd[:D_FEAT]
    Wu2 = W_upd[D_FEAT:]
    b_msg2 = b_msg.reshape(1, D_OUT)
    b_upd2 = b_upd.reshape(1, D_OUT)

    xm = pl.pallas_call(
        _xm_body,
        out_shape=jax.ShapeDtypeStruct((N_NODES, D_OUT), jnp.float32),
    )(x, W1, b_msg2)

    EB = 16000
    em = pl.pallas_call(
        _em_body,
        grid=(N_EDGES // EB,),
        in_specs=[
            pl.BlockSpec((D_EDGE, EB), lambda i: (0, i)),
            pl.BlockSpec((D_EDGE, D_OUT), lambda i: (0, 0)),
        ],
        out_specs=pl.BlockSpec((EB, D_OUT // 2), lambda i: (i, 0)),
        out_shape=jax.ShapeDtypeStruct((N_EDGES, D_OUT // 2), jnp.int32),
    )(edge_attr.T, W2)

    mesh = plsc.VectorSubcoreMesh(
        core_axis_name="c", subcore_axis_name="s", num_cores=NC, num_subcores=NS
    )
    agg2 = pl.kernel(
        _sc_body,
        out_type=jax.ShapeDtypeStruct((NC, N_PAD, D_OUT), jnp.float32),
        mesh=mesh,
        scratch_types=[
            pltpu.VMEM((GPC * CHUNK,), jnp.int32),
            pltpu.VMEM((CHUNK,), jnp.int32),
            pltpu.VMEM((CHUNK,), jnp.int32),
            pltpu.VMEM((CHUNK, D_OUT), jnp.float32),
            pltpu.VMEM((CHUNK, D_OUT), jnp.float32),
            pltpu.VMEM((CHUNK, D_OUT // 2), jnp.int32),
            pltpu.VMEM((CHUNK, D_OUT // 2), jnp.int32),
            pltpu.VMEM_SHARED((N_PAD, D_OUT), jnp.float32),
            pltpu.SemaphoreType.DMA,
            pltpu.SemaphoreType.DMA,
            pltpu.SemaphoreType.DMA,
            pltpu.SemaphoreType.DMA,
            pltpu.SemaphoreType.DMA,
            pltpu.SemaphoreType.DMA,
        ],
    )(xm, em, src, dst)

    NB = 2000
    out = pl.pallas_call(
        _upd_body,
        grid=(N_NODES // NB,),
        in_specs=[
            pl.BlockSpec((NB, D_FEAT), lambda i: (i, 0)),
            pl.BlockSpec((NC, NB, D_OUT), lambda i: (0, i, 0)),
            pl.BlockSpec((D_FEAT, D_OUT), lambda i: (0, 0)),
            pl.BlockSpec((D_OUT, D_OUT), lambda i: (0, 0)),
            pl.BlockSpec((1, D_OUT), lambda i: (0, 0)),
        ],
        out_specs=pl.BlockSpec((NB, D_OUT), lambda i: (i, 0)),
        out_shape=jax.ShapeDtypeStruct((N_NODES, D_OUT), jnp.float32),
    )(x, agg2, Wu1, Wu2, b_upd2)
    return out


def kernel(x, edge_index, edge_attr, W_msg, b_msg, W_upd, b_upd):
    src = edge_index[0].astype(jnp.int32)
    dst = edge_index[1].astype(jnp.int32)
    return _run(x, src, dst, edge_attr, W_msg, b_msg, W_upd, b_upd)
